# Initial kernel scaffold; baseline (speedup 1.0000x reference)
import jax
import jax.numpy as jnp
from jax.experimental import pallas as pl


def kernel(features, edge_index, W_sg, b_sg, W_out, b_out):
    raise NotImplementedError("write your pallas kernel here")


# speedup vs baseline: 8.0382x; 8.0382x over previous
import functools

import jax
import jax.numpy as jnp
from jax import lax
from jax.experimental import pallas as pl
from jax.experimental.pallas import tpu as pltpu
from jax.experimental.pallas import tpu_sc as plsc

N_NODES = 10000
N_PAD = 10240
E_EDGES = 320000
D = 128
C_OUT = 40
NC, NS = 2, 16
NW = NC * NS
EPT = E_EDGES // NW
CHUNK = 125
NCHUNK = EPT // CHUNK
RPS_PAD = N_PAD // NS
WIN_OFF = 624
WIN_LEN = 640
ZR = 64

_mesh = plsc.VectorSubcoreMesh(core_axis_name="c", subcore_axis_name="s")


@functools.partial(
    pl.kernel,
    mesh=_mesh,
    out_type=jax.ShapeDtypeStruct((NC, N_PAD), jnp.float32),
    scratch_types=[
        pltpu.VMEM_SHARED((N_PAD,), jnp.float32),
        pltpu.VMEM((NCHUNK, CHUNK), jnp.int32),
        pltpu.VMEM((128,), jnp.float32),
        pltpu.VMEM((RPS_PAD,), jnp.float32),
    ],
)
def _deg_call(dst_hbm, degp_hbm, acc, dstv, ones_v, zv):
    c = lax.axis_index("c")
    s = lax.axis_index("s")
    w = c * NS + s

    def fill_ones(i, carry):
        ones_v[pl.ds(i * 16, 16)] = jnp.ones((16,), jnp.float32)
        return carry

    lax.fori_loop(0, 128 // 16, fill_ones, 0)

    def fill_z(i, carry):
        zv[pl.ds(i * 16, 16)] = jnp.zeros((16,), jnp.float32)
        return carry

    lax.fori_loop(0, RPS_PAD // 16, fill_z, 0)

    pltpu.sync_copy(zv, acc.at[pl.ds(s * RPS_PAD, RPS_PAD)])
    plsc.subcore_barrier()

    pltpu.sync_copy(dst_hbm.at[pl.ds(w * NCHUNK, NCHUNK)], dstv)

    def body(j, carry):
        pltpu.sync_copy(ones_v.at[pl.ds(0, CHUNK)], acc.at[dstv.at[j]], add=True)
        return carry

    lax.fori_loop(0, NCHUNK, body, 0)
    plsc.subcore_barrier()
    pltpu.sync_copy(acc.at[pl.ds(s * RPS_PAD, RPS_PAD)],
                    degp_hbm.at[c, pl.ds(s * RPS_PAD, RPS_PAD)])


@functools.partial(
    pl.kernel,
    mesh=_mesh,
    out_type=jax.ShapeDtypeStruct((NC, N_NODES, D), jnp.float32),
    scratch_types=[
        pltpu.VMEM_SHARED((N_NODES, D), jnp.float32),
        pltpu.VMEM((NCHUNK, CHUNK), jnp.int32),
        pltpu.VMEM((NCHUNK, CHUNK), jnp.int32),
        pltpu.VMEM((CHUNK, D), jnp.float32),
        pltpu.VMEM((ZR, D), jnp.float32),
        pltpu.SemaphoreType.DMA,
    ],
)
def _hop_call(x_hbm, src_hbm, dst_hbm, outp_hbm, acc, srcv, dstv, rows, zbuf, sem):
    c = lax.axis_index("c")
    s = lax.axis_index("s")
    w = c * NS + s

    def fill_z(i, carry):
        zbuf[i // 8, pl.ds((i % 8) * 16, 16)] = jnp.zeros((16,), jnp.float32)
        return carry

    lax.fori_loop(0, ZR * 8, fill_z, 0)

    def zero_acc(q, carry):
        pltpu.sync_copy(zbuf, acc.at[pl.ds(s * WIN_OFF + q * ZR, ZR)])
        return carry

    lax.fori_loop(0, WIN_LEN // ZR, zero_acc, 0)
    plsc.subcore_barrier()

    pltpu.sync_copy(src_hbm.at[pl.ds(w * NCHUNK, NCHUNK)], srcv)
    pltpu.sync_copy(dst_hbm.at[pl.ds(w * NCHUNK, NCHUNK)], dstv)

    def body(j, carry):
        pltpu.async_copy(x_hbm.at[srcv.at[j]], rows, sem).wait()
        pltpu.sync_copy(rows, acc.at[dstv.at[j]], add=True)
        return carry

    lax.fori_loop(0, NCHUNK, body, 0)
    plsc.subcore_barrier()
    pltpu.sync_copy(acc.at[pl.ds(s * WIN_OFF, WIN_LEN)],
                    outp_hbm.at[c, pl.ds(s * WIN_OFF, WIN_LEN)])


R = 2000
GRID = N_NODES // R


def _scale_feat_body(degp_ref, feat_ref, x1_ref):
    deg = degp_ref[0, 0, :] + degp_ref[0, 1, :]
    norm = lax.rsqrt(jnp.maximum(deg, 1.0))
    x1_ref[...] = feat_ref[...] * norm[:, None]


def _combine_mid_body(degp_ref, yp_ref, x2_ref):
    deg = degp_ref[0, 0, :] + degp_ref[0, 1, :]
    dinv = 1.0 / jnp.maximum(deg, 1.0)
    x2_ref[...] = (yp_ref[0] + yp_ref[1]) * dinv[:, None]


def _head_body(degp_ref, yp_ref, wsg_ref, bsg_ref, wout_ref, bout_ref, out_ref):
    deg = degp_ref[0, 0, :] + degp_ref[0, 1, :]
    norm = lax.rsqrt(jnp.maximum(deg, 1.0))
    h = (yp_ref[0] + yp_ref[1]) * norm[:, None]
    g = jnp.dot(h, wsg_ref[...], preferred_element_type=jnp.float32) + bsg_ref[...]
    g = jnp.maximum(g, 0.0)
    out_ref[...] = (jnp.dot(g, wout_ref[...], preferred_element_type=jnp.float32)
                    + bout_ref[...])


_deg_spec = pl.BlockSpec((1, 2, R), lambda i: (i, 0, 0))
_row_spec = pl.BlockSpec((R, D), lambda i: (i, 0))
_part_spec = pl.BlockSpec((2, R, D), lambda i: (0, i, 0))


def kernel(features, edge_index, W_sg, b_sg, W_out, b_out):
    src2 = edge_index[0].reshape(E_EDGES // CHUNK, CHUNK)
    dst2 = edge_index[1].reshape(E_EDGES // CHUNK, CHUNK)

    degp = _deg_call(dst2)
    degp3 = degp[:, :N_NODES].reshape(2, GRID, R).transpose(1, 0, 2)

    x1 = pl.pallas_call(
        _scale_feat_body,
        grid=(GRID,),
        in_specs=[_deg_spec, _row_spec],
        out_specs=_row_spec,
        out_shape=jax.ShapeDtypeStruct((N_NODES, D), jnp.float32),
    )(degp3, features)

    y1p = _hop_call(x1, src2, dst2)

    x2 = pl.pallas_call(
        _combine_mid_body,
        grid=(GRID,),
        in_specs=[_deg_spec, _part_spec],
        out_specs=_row_spec,
        out_shape=jax.ShapeDtypeStruct((N_NODES, D), jnp.float32),
    )(degp3, y1p)

    y2p = _hop_call(x2, src2, dst2)

    out = pl.pallas_call(
        _head_body,
        grid=(GRID,),
        in_specs=[
            _deg_spec,
            _part_spec,
            pl.BlockSpec((D, D), lambda i: (0, 0)),
            pl.BlockSpec((1, D), lambda i: (0, 0)),
            pl.BlockSpec((D, C_OUT), lambda i: (0, 0)),
            pl.BlockSpec((1, C_OUT), lambda i: (0, 0)),
        ],
        out_specs=pl.BlockSpec((R, C_OUT), lambda i: (i, 0)),
        out_shape=jax.ShapeDtypeStruct((N_NODES, C_OUT), jnp.float32),
    )(degp3, y2p, W_sg, b_sg.reshape(1, D), W_out, b_out.reshape(1, C_OUT))
    return out


# speedup vs baseline: 11.5201x; 1.4332x over previous
import functools

import jax
import jax.numpy as jnp
from jax import lax
from jax.experimental import pallas as pl
from jax.experimental.pallas import tpu as pltpu
from jax.experimental.pallas import tpu_sc as plsc

N_NODES = 10000
N_PAD = 10240
E_EDGES = 320000
D = 128
C_OUT = 40
NC, NS = 2, 16
NW = NC * NS
EPT = E_EDGES // NW
CHUNK = 125
NCHUNK = EPT // CHUNK
RPS_PAD = N_PAD // NS
WIN_OFF = 624
WIN_LEN = 640
ZR = 64

_mesh = plsc.VectorSubcoreMesh(core_axis_name="c", subcore_axis_name="s")


@functools.partial(
    pl.kernel,
    mesh=_mesh,
    out_type=jax.ShapeDtypeStruct((NC, N_PAD), jnp.float32),
    scratch_types=[
        pltpu.VMEM_SHARED((N_PAD,), jnp.float32),
        pltpu.VMEM((NCHUNK, CHUNK), jnp.int32),
        pltpu.VMEM((128,), jnp.float32),
        pltpu.VMEM((RPS_PAD,), jnp.float32),
    ],
)
def _deg_call(dst_hbm, degp_hbm, acc, dstv, ones_v, zv):
    c = lax.axis_index("c")
    s = lax.axis_index("s")
    w = c * NS + s

    def fill_ones(i, carry):
        ones_v[pl.ds(i * 16, 16)] = jnp.ones((16,), jnp.float32)
        return carry

    lax.fori_loop(0, 128 // 16, fill_ones, 0)

    def fill_z(i, carry):
        zv[pl.ds(i * 16, 16)] = jnp.zeros((16,), jnp.float32)
        return carry

    lax.fori_loop(0, RPS_PAD // 16, fill_z, 0)

    pltpu.sync_copy(zv, acc.at[pl.ds(s * RPS_PAD, RPS_PAD)])
    plsc.subcore_barrier()

    pltpu.sync_copy(dst_hbm.at[pl.ds(w * NCHUNK, NCHUNK)], dstv)

    def body(j, carry):
        pltpu.sync_copy(ones_v.at[pl.ds(0, CHUNK)], acc.at[dstv.at[j]], add=True)
        return carry

    lax.fori_loop(0, NCHUNK, body, 0)
    plsc.subcore_barrier()
    pltpu.sync_copy(acc.at[pl.ds(s * RPS_PAD, RPS_PAD)],
                    degp_hbm.at[c, pl.ds(s * RPS_PAD, RPS_PAD)])


PCH = NCHUNK // 2
PAIRS = PCH // 2


@functools.partial(
    pl.kernel,
    mesh=_mesh,
    out_type=jax.ShapeDtypeStruct((NC, N_NODES, D), jnp.float32),
    scratch_types=[
        pltpu.VMEM_SHARED((N_NODES, D), jnp.float32),
        pltpu.VMEM((PCH, CHUNK), jnp.int32),
        pltpu.VMEM((PCH, CHUNK), jnp.int32),
        pltpu.VMEM((CHUNK, D), jnp.float32),
        pltpu.VMEM((CHUNK, D), jnp.float32),
        pltpu.SemaphoreType.DMA,
        pltpu.SemaphoreType.DMA,
    ],
)
def _hop_call(x_hbm, src_hbm, dst_hbm, outp_hbm, acc, srcv, dstv,
              rows0, rows1, sem0, sem1):
    c = lax.axis_index("c")
    s = lax.axis_index("s")
    w = c * NS + s

    def fill_z(i, carry):
        rows0[i // 8, pl.ds((i % 8) * 16, 16)] = jnp.zeros((16,), jnp.float32)
        return carry

    lax.fori_loop(0, CHUNK * 8, fill_z, 0)

    def zero_acc(q, carry):
        pltpu.sync_copy(rows0.at[pl.ds(0, 80)],
                        acc.at[pl.ds(s * WIN_OFF + q * 80, 80)])
        return carry

    lax.fori_loop(0, WIN_LEN // 80, zero_acc, 0)
    plsc.subcore_barrier()

    for h in range(NCHUNK // PCH):
        pltpu.sync_copy(src_hbm.at[pl.ds(w * NCHUNK + h * PCH, PCH)], srcv)
        pltpu.sync_copy(dst_hbm.at[pl.ds(w * NCHUNK + h * PCH, PCH)], dstv)
        pltpu.async_copy(x_hbm.at[srcv.at[0]], rows0, sem0)

        def pair(t, carry):
            a = 2 * t
            b = 2 * t + 1
            hb = pltpu.async_copy(x_hbm.at[srcv.at[b]], rows1, sem1)
            pltpu.make_async_copy(x_hbm.at[srcv.at[a]], rows0, sem0).wait()
            pltpu.sync_copy(rows0, acc.at[dstv.at[a]], add=True)

            @pl.when(t < PAIRS - 1)
            def _():
                pltpu.async_copy(x_hbm.at[srcv.at[a + 2]], rows0, sem0)

            hb.wait()
            pltpu.sync_copy(rows1, acc.at[dstv.at[b]], add=True)
            return carry

        lax.fori_loop(0, PAIRS, pair, 0)

    plsc.subcore_barrier()
    pltpu.sync_copy(acc.at[pl.ds(s * WIN_OFF, WIN_LEN)],
                    outp_hbm.at[c, pl.ds(s * WIN_OFF, WIN_LEN)])


R = 2000
GRID = N_NODES // R


def _scale_feat_body(degp_ref, feat_ref, x1_ref):
    deg = degp_ref[0, 0, :] + degp_ref[0, 1, :]
    norm = lax.rsqrt(jnp.maximum(deg, 1.0))
    x1_ref[...] = feat_ref[...] * norm[:, None]


def _combine_mid_body(degp_ref, yp_ref, x2_ref):
    deg = degp_ref[0, 0, :] + degp_ref[0, 1, :]
    dinv = 1.0 / jnp.maximum(deg, 1.0)
    x2_ref[...] = (yp_ref[0] + yp_ref[1]) * dinv[:, None]


def _head_body(degp_ref, yp_ref, wsg_ref, bsg_ref, wout_ref, bout_ref, out_ref):
    deg = degp_ref[0, 0, :] + degp_ref[0, 1, :]
    norm = lax.rsqrt(jnp.maximum(deg, 1.0))
    h = (yp_ref[0] + yp_ref[1]) * norm[:, None]
    g = jnp.dot(h, wsg_ref[...], preferred_element_type=jnp.float32) + bsg_ref[...]
    g = jnp.maximum(g, 0.0)
    out_ref[...] = (jnp.dot(g, wout_ref[...], preferred_element_type=jnp.float32)
                    + bout_ref[...])


_deg_spec = pl.BlockSpec((1, 2, R), lambda i: (i, 0, 0))
_row_spec = pl.BlockSpec((R, D), lambda i: (i, 0))
_part_spec = pl.BlockSpec((2, R, D), lambda i: (0, i, 0))


def kernel(features, edge_index, W_sg, b_sg, W_out, b_out):
    src2 = edge_index[0].reshape(E_EDGES // CHUNK, CHUNK)
    dst2 = edge_index[1].reshape(E_EDGES // CHUNK, CHUNK)

    degp = _deg_call(dst2)
    degp3 = degp[:, :N_NODES].reshape(2, GRID, R).transpose(1, 0, 2)

    x1 = pl.pallas_call(
        _scale_feat_body,
        grid=(GRID,),
        in_specs=[_deg_spec, _row_spec],
        out_specs=_row_spec,
        out_shape=jax.ShapeDtypeStruct((N_NODES, D), jnp.float32),
    )(degp3, features)

    y1p = _hop_call(x1, src2, dst2)

    x2 = pl.pallas_call(
        _combine_mid_body,
        grid=(GRID,),
        in_specs=[_deg_spec, _part_spec],
        out_specs=_row_spec,
        out_shape=jax.ShapeDtypeStruct((N_NODES, D), jnp.float32),
    )(degp3, y1p)

    y2p = _hop_call(x2, src2, dst2)

    out = pl.pallas_call(
        _head_body,
        grid=(GRID,),
        in_specs=[
            _deg_spec,
            _part_spec,
            pl.BlockSpec((D, D), lambda i: (0, 0)),
            pl.BlockSpec((1, D), lambda i: (0, 0)),
            pl.BlockSpec((D, C_OUT), lambda i: (0, 0)),
            pl.BlockSpec((1, C_OUT), lambda i: (0, 0)),
        ],
        out_specs=pl.BlockSpec((R, C_OUT), lambda i: (i, 0)),
        out_shape=jax.ShapeDtypeStruct((N_NODES, C_OUT), jnp.float32),
    )(degp3, y2p, W_sg, b_sg.reshape(1, D), W_out, b_out.reshape(1, C_OUT))
    return out


# speedup vs baseline: 11.8013x; 1.0244x over previous
import functools

import jax
import jax.numpy as jnp
from jax import lax
from jax.experimental import pallas as pl
from jax.experimental.pallas import tpu as pltpu
from jax.experimental.pallas import tpu_sc as plsc

N_NODES = 10000
N_PAD = 10240
E_EDGES = 320000
D = 128
C_OUT = 40
NC, NS = 2, 16
NW = NC * NS
EPT = E_EDGES // NW
CHUNK = 125
NCHUNK = EPT // CHUNK
RPS_PAD = N_PAD // NS
WIN_OFF = 624
WIN_LEN = 640
ZR = 64

_mesh = plsc.VectorSubcoreMesh(core_axis_name="c", subcore_axis_name="s")


@functools.partial(
    pl.kernel,
    mesh=_mesh,
    out_type=jax.ShapeDtypeStruct((NC, N_PAD), jnp.float32),
    scratch_types=[
        pltpu.VMEM_SHARED((N_PAD,), jnp.float32),
        pltpu.VMEM((NCHUNK, CHUNK), jnp.int32),
        pltpu.VMEM((128,), jnp.float32),
        pltpu.VMEM((RPS_PAD,), jnp.float32),
        pltpu.SemaphoreType.DMA,
    ],
)
def _deg_call(dst_hbm, degp_hbm, acc, dstv, ones_v, zv, dsem):
    c = lax.axis_index("c")
    s = lax.axis_index("s")
    w = c * NS + s

    def fill_ones(i, carry):
        ones_v[pl.ds(i * 16, 16)] = jnp.ones((16,), jnp.float32)
        return carry

    lax.fori_loop(0, 128 // 16, fill_ones, 0)

    def fill_z(i, carry):
        zv[pl.ds(i * 16, 16)] = jnp.zeros((16,), jnp.float32)
        return carry

    lax.fori_loop(0, RPS_PAD // 16, fill_z, 0)

    pltpu.sync_copy(zv, acc.at[pl.ds(s * RPS_PAD, RPS_PAD)])
    plsc.subcore_barrier()

    pltpu.sync_copy(dst_hbm.at[pl.ds(w * NCHUNK, NCHUNK)], dstv)

    def group(g, carry):
        for k in range(16):
            pltpu.async_copy(ones_v.at[pl.ds(0, CHUNK)],
                             acc.at[dstv.at[g * 16 + k]], dsem, add=True)
        for k in range(16):
            pltpu.make_async_copy(ones_v.at[pl.ds(0, CHUNK)],
                                  acc.at[dstv.at[g * 16 + k]], dsem).wait()
        return carry

    lax.fori_loop(0, NCHUNK // 16, group, 0)
    plsc.subcore_barrier()
    pltpu.sync_copy(acc.at[pl.ds(s * RPS_PAD, RPS_PAD)],
                    degp_hbm.at[c, pl.ds(s * RPS_PAD, RPS_PAD)])


PCH = NCHUNK // 2
PAIRS = PCH // 2


@functools.partial(
    pl.kernel,
    mesh=_mesh,
    out_type=jax.ShapeDtypeStruct((NC, N_NODES, D), jnp.float32),
    scratch_types=[
        pltpu.VMEM_SHARED((N_NODES, D), jnp.float32),
        pltpu.VMEM((PCH, CHUNK), jnp.int32),
        pltpu.VMEM((PCH, CHUNK), jnp.int32),
        pltpu.VMEM((CHUNK, D), jnp.float32),
        pltpu.VMEM((CHUNK, D), jnp.float32),
        pltpu.SemaphoreType.DMA,
        pltpu.SemaphoreType.DMA,
    ],
)
def _hop_call(x_hbm, src_hbm, dst_hbm, outp_hbm, acc, srcv, dstv,
              rows0, rows1, sem0, sem1):
    c = lax.axis_index("c")
    s = lax.axis_index("s")
    w = c * NS + s

    pltpu.sync_copy(src_hbm.at[pl.ds(w * NCHUNK, PCH)], srcv)
    pltpu.sync_copy(dst_hbm.at[pl.ds(w * NCHUNK, PCH)], dstv)
    pltpu.async_copy(x_hbm.at[srcv.at[0]], rows0, sem0)

    def fill_z(i, carry):
        rows1[i // 8, pl.ds((i % 8) * 16, 16)] = jnp.zeros((16,), jnp.float32)
        return carry

    lax.fori_loop(0, CHUNK * 8, fill_z, 0)

    def zero_acc(q, carry):
        pltpu.sync_copy(rows1.at[pl.ds(0, 80)],
                        acc.at[pl.ds(s * WIN_OFF + q * 80, 80)])
        return carry

    lax.fori_loop(0, WIN_LEN // 80, zero_acc, 0)
    plsc.subcore_barrier()

    for h in range(NCHUNK // PCH):
        if h > 0:
            pltpu.sync_copy(src_hbm.at[pl.ds(w * NCHUNK + h * PCH, PCH)], srcv)
            pltpu.sync_copy(dst_hbm.at[pl.ds(w * NCHUNK + h * PCH, PCH)], dstv)
            pltpu.async_copy(x_hbm.at[srcv.at[0]], rows0, sem0)

        def pair(t, carry):
            a = 2 * t
            b = 2 * t + 1
            hb = pltpu.async_copy(x_hbm.at[srcv.at[b]], rows1, sem1)
            pltpu.make_async_copy(x_hbm.at[srcv.at[a]], rows0, sem0).wait()
            pltpu.sync_copy(rows0, acc.at[dstv.at[a]], add=True)

            @pl.when(t < PAIRS - 1)
            def _():
                pltpu.async_copy(x_hbm.at[srcv.at[a + 2]], rows0, sem0)

            hb.wait()
            pltpu.sync_copy(rows1, acc.at[dstv.at[b]], add=True)
            return carry

        lax.fori_loop(0, PAIRS, pair, 0)

    plsc.subcore_barrier()
    pltpu.sync_copy(acc.at[pl.ds(s * WIN_OFF, WIN_LEN)],
                    outp_hbm.at[c, pl.ds(s * WIN_OFF, WIN_LEN)])


R = 2000
GRID = N_NODES // R


def _scale_feat_body(degp_ref, feat_ref, x1_ref):
    deg = degp_ref[0, 0, :] + degp_ref[0, 1, :]
    norm = lax.rsqrt(jnp.maximum(deg, 1.0))
    x1_ref[...] = feat_ref[...] * norm[:, None]


def _combine_mid_body(degp_ref, yp_ref, x2_ref):
    deg = degp_ref[0, 0, :] + degp_ref[0, 1, :]
    dinv = 1.0 / jnp.maximum(deg, 1.0)
    x2_ref[...] = (yp_ref[0] + yp_ref[1]) * dinv[:, None]


def _head_body(degp_ref, yp_ref, wsg_ref, bsg_ref, wout_ref, bout_ref, out_ref):
    deg = degp_ref[0, 0, :] + degp_ref[0, 1, :]
    norm = lax.rsqrt(jnp.maximum(deg, 1.0))
    h = (yp_ref[0] + yp_ref[1]) * norm[:, None]
    g = jnp.dot(h, wsg_ref[...], preferred_element_type=jnp.float32) + bsg_ref[...]
    g = jnp.maximum(g, 0.0)
    out_ref[...] = (jnp.dot(g, wout_ref[...], preferred_element_type=jnp.float32)
                    + bout_ref[...])


_deg_spec = pl.BlockSpec((1, 2, R), lambda i: (i, 0, 0))
_row_spec = pl.BlockSpec((R, D), lambda i: (i, 0))
_part_spec = pl.BlockSpec((2, R, D), lambda i: (0, i, 0))


def kernel(features, edge_index, W_sg, b_sg, W_out, b_out):
    src2 = edge_index[0].reshape(E_EDGES // CHUNK, CHUNK)
    dst2 = edge_index[1].reshape(E_EDGES // CHUNK, CHUNK)

    degp = _deg_call(dst2)
    degp3 = degp[:, :N_NODES].reshape(2, GRID, R).transpose(1, 0, 2)

    x1 = pl.pallas_call(
        _scale_feat_body,
        grid=(GRID,),
        in_specs=[_deg_spec, _row_spec],
        out_specs=_row_spec,
        out_shape=jax.ShapeDtypeStruct((N_NODES, D), jnp.float32),
    )(degp3, features)

    y1p = _hop_call(x1, src2, dst2)

    x2 = pl.pallas_call(
        _combine_mid_body,
        grid=(GRID,),
        in_specs=[_deg_spec, _part_spec],
        out_specs=_row_spec,
        out_shape=jax.ShapeDtypeStruct((N_NODES, D), jnp.float32),
    )(degp3, y1p)

    y2p = _hop_call(x2, src2, dst2)

    out = pl.pallas_call(
        _head_body,
        grid=(GRID,),
        in_specs=[
            _deg_spec,
            _part_spec,
            pl.BlockSpec((D, D), lambda i: (0, 0)),
            pl.BlockSpec((1, D), lambda i: (0, 0)),
            pl.BlockSpec((D, C_OUT), lambda i: (0, 0)),
            pl.BlockSpec((1, C_OUT), lambda i: (0, 0)),
        ],
        out_specs=pl.BlockSpec((R, C_OUT), lambda i: (i, 0)),
        out_shape=jax.ShapeDtypeStruct((N_NODES, C_OUT), jnp.float32),
    )(degp3, y2p, W_sg, b_sg.reshape(1, D), W_out, b_out.reshape(1, C_OUT))
    return out


# speedup vs baseline: 11.8957x; 1.0080x over previous
import functools

import jax
import jax.numpy as jnp
from jax import lax
from jax.experimental import pallas as pl
from jax.experimental.pallas import tpu as pltpu
from jax.experimental.pallas import tpu_sc as plsc

N_NODES = 10000
N_PAD = 10240
E_EDGES = 320000
D = 128
C_OUT = 40
NC, NS = 2, 16
NW = NC * NS
EPT = E_EDGES // NW
CHUNK = 125
NCHUNK = EPT // CHUNK
RPS_PAD = N_PAD // NS
WIN_OFF = 624
WIN_LEN = 640
ZR = 64

_mesh = plsc.VectorSubcoreMesh(core_axis_name="c", subcore_axis_name="s")


@functools.partial(
    pl.kernel,
    mesh=_mesh,
    out_type=jax.ShapeDtypeStruct((NC, N_PAD), jnp.float32),
    scratch_types=[
        pltpu.VMEM_SHARED((N_PAD,), jnp.float32),
        pltpu.VMEM((NCHUNK, CHUNK), jnp.int32),
        pltpu.VMEM((128,), jnp.float32),
        pltpu.VMEM((RPS_PAD,), jnp.float32),
        pltpu.SemaphoreType.DMA,
    ],
)
def _deg_call(dst_hbm, degp_hbm, acc, dstv, ones_v, zv, dsem):
    c = lax.axis_index("c")
    s = lax.axis_index("s")
    w = c * NS + s

    def fill_ones(i, carry):
        ones_v[pl.ds(i * 16, 16)] = jnp.ones((16,), jnp.float32)
        return carry

    lax.fori_loop(0, 128 // 16, fill_ones, 0)

    def fill_z(i, carry):
        zv[pl.ds(i * 16, 16)] = jnp.zeros((16,), jnp.float32)
        return carry

    lax.fori_loop(0, RPS_PAD // 16, fill_z, 0)

    pltpu.sync_copy(zv, acc.at[pl.ds(s * RPS_PAD, RPS_PAD)])
    plsc.subcore_barrier()

    pltpu.sync_copy(dst_hbm.at[pl.ds(w * NCHUNK, NCHUNK)], dstv)

    def group(g, carry):
        for k in range(16):
            pltpu.async_copy(ones_v.at[pl.ds(0, CHUNK)],
                             acc.at[dstv.at[g * 16 + k]], dsem, add=True)
        for k in range(16):
            pltpu.make_async_copy(ones_v.at[pl.ds(0, CHUNK)],
                                  acc.at[dstv.at[g * 16 + k]], dsem).wait()
        return carry

    lax.fori_loop(0, NCHUNK // 16, group, 0)
    plsc.subcore_barrier()
    pltpu.sync_copy(acc.at[pl.ds(s * RPS_PAD, RPS_PAD)],
                    degp_hbm.at[c, pl.ds(s * RPS_PAD, RPS_PAD)])


PCH = NCHUNK // 2
PAIRS = PCH // 2


@functools.partial(
    pl.kernel,
    mesh=_mesh,
    out_type=jax.ShapeDtypeStruct((NC, N_NODES, D), jnp.float32),
    scratch_types=[
        pltpu.VMEM_SHARED((N_NODES, D), jnp.float32),
        pltpu.VMEM((PCH, CHUNK), jnp.int32),
        pltpu.VMEM((PCH, CHUNK), jnp.int32),
        pltpu.VMEM((CHUNK, D), jnp.float32),
        pltpu.VMEM((CHUNK, D), jnp.float32),
        pltpu.SemaphoreType.DMA,
        pltpu.SemaphoreType.DMA,
    ],
)
def _hop_call(x_hbm, src_hbm, dst_hbm, outp_hbm, acc, srcv, dstv,
              rows0, rows1, sem0, sem1):
    c = lax.axis_index("c")
    s = lax.axis_index("s")
    w = c * NS + s

    pltpu.sync_copy(src_hbm.at[pl.ds(w * NCHUNK, PCH)], srcv)
    pltpu.sync_copy(dst_hbm.at[pl.ds(w * NCHUNK, PCH)], dstv)
    pltpu.async_copy(x_hbm.at[srcv.at[0]], rows0, sem0)

    def fill_z(i, carry):
        rows1[i // 8, pl.ds((i % 8) * 16, 16)] = jnp.zeros((16,), jnp.float32)
        return carry

    lax.fori_loop(0, CHUNK * 8, fill_z, 0)

    def zero_acc(q, carry):
        pltpu.sync_copy(rows1.at[pl.ds(0, 80)],
                        acc.at[pl.ds(s * WIN_OFF + q * 80, 80)])
        return carry

    lax.fori_loop(0, WIN_LEN // 80, zero_acc, 0)
    plsc.subcore_barrier()

    for h in range(NCHUNK // PCH):
        if h > 0:
            pltpu.sync_copy(src_hbm.at[pl.ds(w * NCHUNK + h * PCH, PCH)], srcv)
            pltpu.sync_copy(dst_hbm.at[pl.ds(w * NCHUNK + h * PCH, PCH)], dstv)
            pltpu.async_copy(x_hbm.at[srcv.at[0]], rows0, sem0)

        def pair(t, carry):
            a = 2 * t
            b = 2 * t + 1
            hb = pltpu.async_copy(x_hbm.at[srcv.at[b]], rows1, sem1)
            pltpu.make_async_copy(x_hbm.at[srcv.at[a]], rows0, sem0).wait()
            pltpu.sync_copy(rows0, acc.at[dstv.at[a]], add=True)

            @pl.when(t < PAIRS - 1)
            def _():
                pltpu.async_copy(x_hbm.at[srcv.at[a + 2]], rows0, sem0)

            hb.wait()
            pltpu.sync_copy(rows1, acc.at[dstv.at[b]], add=True)
            return carry

        lax.fori_loop(0, PAIRS, pair, 0)

    plsc.subcore_barrier()
    pltpu.sync_copy(acc.at[pl.ds(s * WIN_OFF, WIN_LEN)],
                    outp_hbm.at[c, pl.ds(s * WIN_OFF, WIN_LEN)])


R = 2048
GRID = (N_NODES + R - 1) // R


def _deg_slice(degp_ref):
    i = pl.program_id(0)
    return degp_ref[0, pl.ds(i * R, R)] + degp_ref[1, pl.ds(i * R, R)]


def _scale_feat_body(degp_ref, feat_ref, x1_ref):
    deg = _deg_slice(degp_ref)
    norm = lax.rsqrt(jnp.maximum(deg, 1.0))
    x1_ref[...] = feat_ref[...] * norm[:, None]


def _combine_mid_body(degp_ref, yp_ref, x2_ref):
    deg = _deg_slice(degp_ref)
    dinv = 1.0 / jnp.maximum(deg, 1.0)
    x2_ref[...] = (yp_ref[0] + yp_ref[1]) * dinv[:, None]


def _head_body(degp_ref, yp_ref, wsg_ref, bsg_ref, wout_ref, bout_ref, out_ref):
    deg = _deg_slice(degp_ref)
    norm = lax.rsqrt(jnp.maximum(deg, 1.0))
    h = (yp_ref[0] + yp_ref[1]) * norm[:, None]
    g = jnp.dot(h, wsg_ref[...], preferred_element_type=jnp.float32) + bsg_ref[...]
    g = jnp.maximum(g, 0.0)
    out_ref[...] = (jnp.dot(g, wout_ref[...], preferred_element_type=jnp.float32)
                    + bout_ref[...])


_deg_spec = pl.BlockSpec((NC, N_PAD), lambda i: (0, 0))
_row_spec = pl.BlockSpec((R, D), lambda i: (i, 0))
_part_spec = pl.BlockSpec((2, R, D), lambda i: (0, i, 0))


def kernel(features, edge_index, W_sg, b_sg, W_out, b_out):
    src2 = edge_index[0].reshape(E_EDGES // CHUNK, CHUNK)
    dst2 = edge_index[1].reshape(E_EDGES // CHUNK, CHUNK)

    degp = _deg_call(dst2)

    x1 = pl.pallas_call(
        _scale_feat_body,
        grid=(GRID,),
        in_specs=[_deg_spec, _row_spec],
        out_specs=_row_spec,
        out_shape=jax.ShapeDtypeStruct((N_NODES, D), jnp.float32),
    )(degp, features)

    y1p = _hop_call(x1, src2, dst2)

    x2 = pl.pallas_call(
        _combine_mid_body,
        grid=(GRID,),
        in_specs=[_deg_spec, _part_spec],
        out_specs=_row_spec,
        out_shape=jax.ShapeDtypeStruct((N_NODES, D), jnp.float32),
    )(degp, y1p)

    y2p = _hop_call(x2, src2, dst2)

    out = pl.pallas_call(
        _head_body,
        grid=(GRID,),
        in_specs=[
            _deg_spec,
            _part_spec,
            pl.BlockSpec((D, D), lambda i: (0, 0)),
            pl.BlockSpec((1, D), lambda i: (0, 0)),
            pl.BlockSpec((D, C_OUT), lambda i: (0, 0)),
            pl.BlockSpec((1, C_OUT), lambda i: (0, 0)),
        ],
        out_specs=pl.BlockSpec((R, C_OUT), lambda i: (i, 0)),
        out_shape=jax.ShapeDtypeStruct((N_NODES, C_OUT), jnp.float32),
    )(degp, y2p, W_sg, b_sg.reshape(1, D), W_out, b_out.reshape(1, C_OUT))
    return out
